# baseline (speedup 1.0000x reference)
import functools

import jax
import jax.numpy as jnp
from jax import lax
from jax.experimental import pallas as pl
from jax.experimental.pallas import tpu as pltpu
from jax.experimental.pallas import tpu_sc as plsc

N = 16384
D = 256
K = 8192
BR = 1024
RH = 128
KC = 2048
COMMIT = 0.25


def _argmin_body(zsq_ref, wsq_ref, z_ref, w_ref, idx_ref):
    z2 = z_ref[...] * 2.0
    m2 = [
        lax.dot_general(z2, w_ref[pl.ds(c * KC, KC), :],
                        (((1,), (1,)), ((), ())),
                        preferred_element_type=jnp.float32)
        for c in range(K // KC)
    ]
    lanes = 128
    jtot = K // lanes
    jper = KC // lanes
    big = jnp.int32(2**30)
    for h in range(BR // RH):
        r0 = h * RH
        zsq = zsq_ref[pl.ds(r0, RH), :]
        run_v = jnp.full((RH, lanes), jnp.inf, jnp.float32)
        run_j = jnp.zeros((RH, lanes), jnp.int32)
        for j in range(jtot):
            a1 = zsq + wsq_ref[:, pl.ds(j * lanes, lanes)]
            jl = j % jper
            d = a1 - m2[j // jper][r0:r0 + RH, jl * lanes:(jl + 1) * lanes]
            upd = d < run_v
            run_v = jnp.where(upd, d, run_v)
            run_j = jnp.where(upd, jnp.int32(j), run_j)
        gmin = jnp.min(run_v, axis=1, keepdims=True)
        lane_iota = lax.broadcasted_iota(jnp.int32, (RH, lanes), 1)
        kc = jnp.where(run_v == gmin, run_j * lanes + lane_iota, big)
        idx_ref[pl.ds(r0, RH), :] = jnp.min(kc, axis=1, keepdims=True)


def _argmin_call(zsq, wsq2d, z_e, W, nrows, off):
    ob = off // BR
    return pl.pallas_call(
        _argmin_body,
        grid=(nrows // BR,),
        in_specs=[
            pl.BlockSpec((BR, 1), lambda i: (i + ob, 0)),
            pl.BlockSpec((1, K), lambda i: (0, 0)),
            pl.BlockSpec((BR, D), lambda i: (i + ob, 0)),
            pl.BlockSpec((K, D), lambda i: (0, 0)),
        ],
        out_specs=pl.BlockSpec((BR, 1), lambda i: (i, 0)),
        out_shape=jax.ShapeDtypeStruct((nrows, 1), jnp.int32),
    )(zsq, wsq2d, z_e, W)


def _sc_fused_call(W, idx_flat, z_e, zeros_k, nrows, off):
    info = plsc.get_sparse_core_info()
    nc, ns = info.num_cores, info.num_subcores
    nw = nc * ns
    bpw = nrows // nw
    ch = 128
    mesh = plsc.VectorSubcoreMesh(core_axis_name="c", subcore_axis_name="s")

    @functools.partial(
        pl.kernel, mesh=mesh,
        out_type=[
            jax.ShapeDtypeStruct((nrows, D), jnp.float32),
            jax.ShapeDtypeStruct((nw, 16), jnp.float32),
            jax.ShapeDtypeStruct((nc, K), jnp.int32),
        ],
        scratch_types=[
            pltpu.VMEM((bpw,), jnp.int32),
            pltpu.VMEM((bpw,), jnp.int32),
            pltpu.VMEM((ch, D), jnp.float32),
            pltpu.VMEM((ch, D), jnp.float32),
            pltpu.VMEM((16,), jnp.float32),
            pltpu.VMEM_SHARED((K,), jnp.int32),
            pltpu.SemaphoreType.DMA,
        ],
    )
    def fused(table_hbm, idx_hbm, z_hbm, zero_hbm,
              qst_hbm, loss_hbm, cnt_hbm,
              idx_v, ones_v, rows_v, z_v, loss_v, cnt_sh, sem):
        cid = lax.axis_index("c")
        sid = lax.axis_index("s")
        wid = sid * nc + cid
        base = wid * bpw
        pltpu.sync_copy(idx_hbm.at[pl.ds(base, bpw)], idx_v)
        one16 = jnp.ones((16,), jnp.int32)
        for b in range(bpw // 16):
            ones_v[pl.ds(b * 16, 16)] = one16

        @pl.when(sid == 0)
        def _zero():
            pltpu.sync_copy(zero_hbm, cnt_sh)

        plsc.subcore_barrier()
        pltpu.sync_copy(ones_v, cnt_sh.at[idx_v], add=True)

        def chunk(c, acc):
            cbase = base + c * ch
            pltpu.async_copy(table_hbm.at[idx_v.at[pl.ds(c * ch, ch)]],
                             rows_v, sem).wait()
            pltpu.sync_copy(z_hbm.at[pl.ds(off + cbase, ch)], z_v)

            def row(r, acc):
                for l in range(D // 16):
                    q = rows_v[r, pl.ds(l * 16, 16)]
                    zz = z_v[r, pl.ds(l * 16, 16)]
                    dd = q - zz
                    rows_v[r, pl.ds(l * 16, 16)] = zz + dd
                    acc = acc + dd * dd
                return acc

            acc = lax.fori_loop(0, ch, row, acc)
            pltpu.sync_copy(rows_v, qst_hbm.at[pl.ds(cbase, ch)])
            return acc

        acc = lax.fori_loop(0, bpw // ch, chunk, jnp.zeros((16,), jnp.float32))
        loss_v[...] = acc
        pltpu.sync_copy(loss_v, loss_hbm.at[wid])
        plsc.subcore_barrier()

        @pl.when(sid == 0)
        def _out():
            pltpu.sync_copy(cnt_sh, cnt_hbm.at[cid])

    return fused(W, idx_flat, z_e, zeros_k)


def _finalize_body(lossp_ref, cnt_ref, loss_ref, perp_ref):
    m = jnp.sum(lossp_ref[...], keepdims=True) / (N * D)
    loss_ref[...] = m + COMMIT * m
    csum = (cnt_ref[0:8, :] + cnt_ref[8:16, :]
            + cnt_ref[16:24, :] + cnt_ref[24:32, :])
    p = csum * (1.0 / N)
    ent = jnp.sum(p * jnp.log(p + 1e-10), keepdims=True)
    perp_ref[...] = jnp.exp(-ent)


def _finalize_call(lossp, cnt):
    return pl.pallas_call(
        _finalize_body,
        grid=(1,),
        in_specs=[
            pl.BlockSpec((8, 128), lambda i: (0, 0)),
            pl.BlockSpec((32, 1024), lambda i: (0, 0)),
        ],
        out_specs=[
            pl.BlockSpec((1, 1), lambda i: (0, 0)),
            pl.BlockSpec((1, 1), lambda i: (0, 0)),
        ],
        out_shape=[
            jax.ShapeDtypeStruct((1, 1), jnp.float32),
            jax.ShapeDtypeStruct((1, 1), jnp.float32),
        ],
    )(lossp, cnt)


def kernel(z_e, W):
    zsq = jnp.sum(z_e ** 2, axis=1, keepdims=True)
    wsq2d = jnp.sum(W ** 2, axis=1).reshape(1, K)
    zeros_k = jnp.zeros((K,), jnp.int32)
    h = N // 2
    idx0 = _argmin_call(zsq, wsq2d, z_e, W, h, 0)
    q0, l0, c0 = _sc_fused_call(W, idx0.reshape(h), z_e, zeros_k, h, 0)
    idx1 = _argmin_call(zsq, wsq2d, z_e, W, h, h)
    q1, l1, c1 = _sc_fused_call(W, idx1.reshape(h), z_e, zeros_k, h, h)
    qst = jnp.concatenate([q0, q1], axis=0)
    lossp = jnp.concatenate([l0, l1], axis=0).reshape(8, 128)
    cnt = jnp.concatenate([c0, c1], axis=0).astype(jnp.float32)
    loss2d, perp2d = _finalize_call(lossp, cnt.reshape(32, 1024))
    return (loss2d[0, 0], qst, perp2d[0, 0])


# speedup vs baseline: 1.0336x; 1.0336x over previous
import functools

import jax
import jax.numpy as jnp
from jax import lax
from jax.experimental import pallas as pl
from jax.experimental.pallas import tpu as pltpu
from jax.experimental.pallas import tpu_sc as plsc

N = 16384
D = 256
K = 8192
BR = 1024
RH = 128
KC = 2048
COMMIT = 0.25


def _argmin_body(zsq_ref, wsq_ref, z_ref, w_ref, idx_ref):
    z2 = z_ref[...] * 2.0
    m2 = [
        lax.dot_general(z2, w_ref[pl.ds(c * KC, KC), :],
                        (((1,), (1,)), ((), ())),
                        preferred_element_type=jnp.float32)
        for c in range(K // KC)
    ]
    lanes = 128
    jtot = K // lanes
    jper = KC // lanes
    big = jnp.int32(2**30)
    for h in range(BR // RH):
        r0 = h * RH
        zsq = zsq_ref[pl.ds(r0, RH), :]
        run_v = jnp.full((RH, lanes), jnp.inf, jnp.float32)
        run_j = jnp.zeros((RH, lanes), jnp.int32)
        for j in range(jtot):
            a1 = zsq + wsq_ref[:, pl.ds(j * lanes, lanes)]
            jl = j % jper
            d = a1 - m2[j // jper][r0:r0 + RH, jl * lanes:(jl + 1) * lanes]
            upd = d < run_v
            run_v = jnp.where(upd, d, run_v)
            run_j = jnp.where(upd, jnp.int32(j), run_j)
        gmin = jnp.min(run_v, axis=1, keepdims=True)
        lane_iota = lax.broadcasted_iota(jnp.int32, (RH, lanes), 1)
        kc = jnp.where(run_v == gmin, run_j * lanes + lane_iota, big)
        idx_ref[pl.ds(r0, RH), :] = jnp.min(kc, axis=1, keepdims=True)


def _argmin_call(zsq, wsq2d, z_e, W):
    return pl.pallas_call(
        _argmin_body,
        grid=(N // BR,),
        in_specs=[
            pl.BlockSpec((BR, 1), lambda i: (i, 0)),
            pl.BlockSpec((1, K), lambda i: (0, 0)),
            pl.BlockSpec((BR, D), lambda i: (i, 0)),
            pl.BlockSpec((K, D), lambda i: (0, 0)),
        ],
        out_specs=pl.BlockSpec((BR, 1), lambda i: (i, 0)),
        out_shape=jax.ShapeDtypeStruct((N, 1), jnp.int32),
    )(zsq, wsq2d, z_e, W)


def _sc_fused_call(W, idx_flat, z_e, zeros_k):
    info = plsc.get_sparse_core_info()
    nc, ns = info.num_cores, info.num_subcores
    nw = nc * ns
    bpw = N // nw
    ch = 128
    mesh = plsc.VectorSubcoreMesh(core_axis_name="c", subcore_axis_name="s")

    @functools.partial(
        pl.kernel, mesh=mesh,
        out_type=[
            jax.ShapeDtypeStruct((N, D), jnp.float32),
            jax.ShapeDtypeStruct((nw, 16), jnp.float32),
            jax.ShapeDtypeStruct((nc, K), jnp.int32),
        ],
        scratch_types=[
            pltpu.VMEM((bpw,), jnp.int32),
            pltpu.VMEM((bpw,), jnp.int32),
            pltpu.VMEM((ch, D), jnp.float32),
            pltpu.VMEM((ch, D), jnp.float32),
            pltpu.VMEM((16,), jnp.float32),
            pltpu.VMEM_SHARED((K,), jnp.int32),
            pltpu.SemaphoreType.DMA,
            pltpu.SemaphoreType.DMA,
        ],
    )
    def fused(table_hbm, idx_hbm, z_hbm, zero_hbm,
              qst_hbm, loss_hbm, cnt_hbm,
              idx_v, ones_v, rows_v, z_v, loss_v, cnt_sh, sem, zsem):
        cid = lax.axis_index("c")
        sid = lax.axis_index("s")
        wid = sid * nc + cid
        base = wid * bpw
        pltpu.sync_copy(idx_hbm.at[pl.ds(base, bpw)], idx_v)
        one16 = jnp.ones((16,), jnp.int32)
        for b in range(bpw // 16):
            ones_v[pl.ds(b * 16, 16)] = one16

        @pl.when(sid == 0)
        def _zero():
            pltpu.sync_copy(zero_hbm, cnt_sh)

        plsc.subcore_barrier()
        pltpu.sync_copy(ones_v, cnt_sh.at[idx_v], add=True)

        def chunk(c, accs):
            cbase = base + c * ch
            zcp = pltpu.async_copy(z_hbm.at[pl.ds(cbase, ch)], z_v, zsem)
            gcp = pltpu.async_copy(table_hbm.at[idx_v.at[pl.ds(c * ch, ch)]],
                                   rows_v, sem)
            gcp.wait()
            zcp.wait()

            def row(r, accs):
                accs = list(accs)
                for l in range(D // 16):
                    q = rows_v[r, pl.ds(l * 16, 16)]
                    zz = z_v[r, pl.ds(l * 16, 16)]
                    dd = q - zz
                    rows_v[r, pl.ds(l * 16, 16)] = zz + dd
                    accs[l % 8] = accs[l % 8] + dd * dd
                return tuple(accs)

            accs = lax.fori_loop(0, ch, row, accs)
            pltpu.sync_copy(rows_v, qst_hbm.at[pl.ds(cbase, ch)])
            return accs

        accs = lax.fori_loop(0, bpw // ch, chunk,
                             tuple(jnp.zeros((16,), jnp.float32)
                                   for _ in range(8)))
        acc = accs[0]
        for t in accs[1:]:
            acc = acc + t
        loss_v[...] = acc
        pltpu.sync_copy(loss_v, loss_hbm.at[wid])
        plsc.subcore_barrier()

        @pl.when(sid == 0)
        def _out():
            pltpu.sync_copy(cnt_sh, cnt_hbm.at[cid])

    return fused(W, idx_flat, z_e, zeros_k)


def _finalize_body(lossp_ref, cnt_ref, loss_ref, perp_ref):
    m = jnp.sum(lossp_ref[...], keepdims=True) / (N * D)
    loss_ref[...] = m + COMMIT * m
    csum = cnt_ref[0:8, :] + cnt_ref[8:16, :]
    p = csum * (1.0 / N)
    ent = jnp.sum(p * jnp.log(p + 1e-10), keepdims=True)
    perp_ref[...] = jnp.exp(-ent)


def _finalize_call(lossp, cnt):
    return pl.pallas_call(
        _finalize_body,
        grid=(1,),
        in_specs=[
            pl.BlockSpec((4, 128), lambda i: (0, 0)),
            pl.BlockSpec((16, 1024), lambda i: (0, 0)),
        ],
        out_specs=[
            pl.BlockSpec((1, 1), lambda i: (0, 0)),
            pl.BlockSpec((1, 1), lambda i: (0, 0)),
        ],
        out_shape=[
            jax.ShapeDtypeStruct((1, 1), jnp.float32),
            jax.ShapeDtypeStruct((1, 1), jnp.float32),
        ],
    )(lossp, cnt)


def kernel(z_e, W):
    zsq = jnp.sum(z_e ** 2, axis=1, keepdims=True)
    wsq2d = jnp.sum(W ** 2, axis=1).reshape(1, K)
    idx2d = _argmin_call(zsq, wsq2d, z_e, W)
    zeros_k = jnp.zeros((K,), jnp.int32)
    qst, lossp, cnt = _sc_fused_call(W, idx2d.reshape(N), z_e, zeros_k)
    loss2d, perp2d = _finalize_call(
        lossp.reshape(4, 128), cnt.astype(jnp.float32).reshape(16, 1024))
    return (loss2d[0, 0], qst, perp2d[0, 0])


# speedup vs baseline: 1.0950x; 1.0594x over previous
import functools

import jax
import jax.numpy as jnp
from jax import lax
from jax.experimental import pallas as pl
from jax.experimental.pallas import tpu as pltpu
from jax.experimental.pallas import tpu_sc as plsc

N = 16384
D = 256
K = 8192
BR = 1024
RH = 128
KC = 2048
COMMIT = 0.25


def _argmin_body(zsq_ref, wsq_ref, z_ref, w_ref, idx_ref):
    z2 = z_ref[...] * 2.0
    m2 = [
        lax.dot_general(z2, w_ref[pl.ds(c * KC, KC), :],
                        (((1,), (1,)), ((), ())),
                        preferred_element_type=jnp.float32)
        for c in range(K // KC)
    ]
    lanes = 128
    jtot = K // lanes
    jper = KC // lanes
    big = jnp.int32(2**30)
    for h in range(BR // RH):
        r0 = h * RH
        zsq = zsq_ref[pl.ds(r0, RH), :]
        run_v = jnp.full((RH, lanes), jnp.inf, jnp.float32)
        run_j = jnp.zeros((RH, lanes), jnp.int32)
        for j in range(jtot):
            a1 = zsq + wsq_ref[:, pl.ds(j * lanes, lanes)]
            jl = j % jper
            d = a1 - m2[j // jper][r0:r0 + RH, jl * lanes:(jl + 1) * lanes]
            upd = d < run_v
            run_v = jnp.where(upd, d, run_v)
            run_j = jnp.where(upd, jnp.int32(j), run_j)
        gmin = jnp.min(run_v, axis=1, keepdims=True)
        lane_iota = lax.broadcasted_iota(jnp.int32, (RH, lanes), 1)
        kc = jnp.where(run_v == gmin, run_j * lanes + lane_iota, big)
        idx_ref[pl.ds(r0, RH), :] = jnp.min(kc, axis=1, keepdims=True)


def _argmin_call(zsq, wsq2d, z_e, W):
    return pl.pallas_call(
        _argmin_body,
        grid=(N // BR,),
        in_specs=[
            pl.BlockSpec((BR, 1), lambda i: (i, 0)),
            pl.BlockSpec((1, K), lambda i: (0, 0)),
            pl.BlockSpec((BR, D), lambda i: (i, 0)),
            pl.BlockSpec((K, D), lambda i: (0, 0)),
        ],
        out_specs=pl.BlockSpec((BR, 1), lambda i: (i, 0)),
        out_shape=jax.ShapeDtypeStruct((N, 1), jnp.int32),
    )(zsq, wsq2d, z_e, W)


def _sc_fused_call(W, idx_flat, z_e, zeros_k):
    info = plsc.get_sparse_core_info()
    nc, ns = info.num_cores, info.num_subcores
    nw = nc * ns
    bpw = N // nw
    ch = 64
    nch = bpw // ch
    mesh = plsc.VectorSubcoreMesh(core_axis_name="c", subcore_axis_name="s")

    @functools.partial(
        pl.kernel, mesh=mesh,
        out_type=[
            jax.ShapeDtypeStruct((N, D), jnp.float32),
            jax.ShapeDtypeStruct((nw, 16), jnp.float32),
            jax.ShapeDtypeStruct((nc, K), jnp.int32),
        ],
        scratch_types=[
            pltpu.VMEM((bpw,), jnp.int32),
            pltpu.VMEM((bpw,), jnp.int32),
            pltpu.VMEM((ch, D), jnp.float32),
            pltpu.VMEM((ch, D), jnp.float32),
            pltpu.VMEM((ch, D), jnp.float32),
            pltpu.VMEM((ch, D), jnp.float32),
            pltpu.VMEM((16,), jnp.float32),
            pltpu.VMEM_SHARED((K,), jnp.int32),
            pltpu.SemaphoreType.DMA,
            pltpu.SemaphoreType.DMA,
        ],
    )
    def fused(table_hbm, idx_hbm, z_hbm, zero_hbm,
              qst_hbm, loss_hbm, cnt_hbm,
              idx_v, ones_v, rows_v0, z_v0, rows_v1, z_v1,
              loss_v, cnt_sh, sem, zsem):
        cid = lax.axis_index("c")
        sid = lax.axis_index("s")
        wid = sid * nc + cid
        base = wid * bpw
        pltpu.sync_copy(idx_hbm.at[pl.ds(base, bpw)], idx_v)
        one16 = jnp.ones((16,), jnp.int32)
        for b in range(bpw // 16):
            ones_v[pl.ds(b * 16, 16)] = one16

        @pl.when(sid == 0)
        def _zero():
            pltpu.sync_copy(zero_hbm, cnt_sh)

        plsc.subcore_barrier()
        pltpu.sync_copy(ones_v, cnt_sh.at[idx_v], add=True)

        def start_in(g, rb, zb):
            pltpu.async_copy(z_hbm.at[pl.ds(base + g * ch, ch)], zb, zsem)
            pltpu.async_copy(table_hbm.at[idx_v.at[pl.ds(g * ch, ch)]],
                             rb, sem)

        def wait_in(g, rb, zb):
            pltpu.make_async_copy(
                table_hbm.at[idx_v.at[pl.ds(g * ch, ch)]], rb, sem).wait()
            pltpu.make_async_copy(
                z_hbm.at[pl.ds(base + g * ch, ch)], zb, zsem).wait()

        def compute(rb, zb, accs):
            def row(r, accs):
                accs = list(accs)
                for l in range(D // 16):
                    q = rb[r, pl.ds(l * 16, 16)]
                    zz = zb[r, pl.ds(l * 16, 16)]
                    dd = q - zz
                    rb[r, pl.ds(l * 16, 16)] = zz + dd
                    accs[l % 8] = accs[l % 8] + dd * dd
                return tuple(accs)

            return lax.fori_loop(0, ch, row, accs)

        start_in(0, rows_v0, z_v0)

        def outer(go, accs):
            for b in (0, 1):
                g = 2 * go + b
                rb, zb = (rows_v0, z_v0) if b == 0 else (rows_v1, z_v1)
                nb, nz = (rows_v1, z_v1) if b == 0 else (rows_v0, z_v0)

                @pl.when(g + 1 < nch)
                def _prefetch():
                    start_in(g + 1, nb, nz)

                wait_in(g, rb, zb)
                accs = compute(rb, zb, accs)
                pltpu.sync_copy(rb, qst_hbm.at[pl.ds(base + g * ch, ch)])
            return accs

        accs = lax.fori_loop(0, nch // 2, outer,
                             tuple(jnp.zeros((16,), jnp.float32)
                                   for _ in range(8)))
        acc = accs[0]
        for t in accs[1:]:
            acc = acc + t
        loss_v[...] = acc
        pltpu.sync_copy(loss_v, loss_hbm.at[wid])
        plsc.subcore_barrier()

        @pl.when(sid == 0)
        def _out():
            pltpu.sync_copy(cnt_sh, cnt_hbm.at[cid])

    return fused(W, idx_flat, z_e, zeros_k)


def _finalize_body(lossp_ref, cnt_ref, loss_ref, perp_ref):
    m = jnp.sum(lossp_ref[...], keepdims=True) / (N * D)
    loss_ref[...] = m + COMMIT * m
    csum = cnt_ref[0:8, :] + cnt_ref[8:16, :]
    p = csum * (1.0 / N)
    ent = jnp.sum(p * jnp.log(p + 1e-10), keepdims=True)
    perp_ref[...] = jnp.exp(-ent)


def _finalize_call(lossp, cnt):
    return pl.pallas_call(
        _finalize_body,
        grid=(1,),
        in_specs=[
            pl.BlockSpec((4, 128), lambda i: (0, 0)),
            pl.BlockSpec((16, 1024), lambda i: (0, 0)),
        ],
        out_specs=[
            pl.BlockSpec((1, 1), lambda i: (0, 0)),
            pl.BlockSpec((1, 1), lambda i: (0, 0)),
        ],
        out_shape=[
            jax.ShapeDtypeStruct((1, 1), jnp.float32),
            jax.ShapeDtypeStruct((1, 1), jnp.float32),
        ],
    )(lossp, cnt)


def kernel(z_e, W):
    zsq = jnp.sum(z_e ** 2, axis=1, keepdims=True)
    wsq2d = jnp.sum(W ** 2, axis=1).reshape(1, K)
    idx2d = _argmin_call(zsq, wsq2d, z_e, W)
    zeros_k = jnp.zeros((K,), jnp.int32)
    qst, lossp, cnt = _sc_fused_call(W, idx2d.reshape(N), z_e, zeros_k)
    loss2d, perp2d = _finalize_call(
        lossp.reshape(4, 128), cnt.astype(jnp.float32).reshape(16, 1024))
    return (loss2d[0, 0], qst, perp2d[0, 0])
